# baseline (speedup 1.0000x reference)
import functools
import math

import jax
import jax.numpy as jnp
from jax import lax
from jax.experimental import pallas as pl
from jax.experimental.pallas import tpu as pltpu
from jax.experimental.pallas import tpu_sc as plsc

VOCAB = 100000
D = 128
BATCH = 4096
SEQ = 50
SEQ_PAD = 56
NW = 32
K_PARTS = 4
PART_B = BATCH // K_PARTS
B_PER_W = PART_B // NW
ROWS_PER_W = B_PER_W * SEQ_PAD
NB = 4
CHUNK = NB * SEQ_PAD
GATHER_SPLITS = ((0, 128), (128, 96))
N_CHUNKS = B_PER_W // NB
NBUF = 4
SCALE = math.sqrt(D)

_mesh = plsc.VectorSubcoreMesh(core_axis_name="c", subcore_axis_name="s")


@functools.partial(
    pl.kernel,
    out_type=jax.ShapeDtypeStruct((PART_B * SEQ_PAD, D), jnp.float32),
    mesh=_mesh,
    scratch_types=[
        pltpu.VMEM((ROWS_PER_W,), jnp.int32),
        pltpu.VMEM((NBUF, CHUNK, D), jnp.float32),
    ]
    + [pltpu.SemaphoreType.DMA] * (2 * NBUF),
)
def _gather_part(idx_hbm, table_hbm, out_hbm, idx_v, gbuf, *sems):
    gsem = sems[:NBUF]
    ssem = sems[NBUF:]
    wid = lax.axis_index("s") * 2 + lax.axis_index("c")
    base = wid * ROWS_PER_W
    pltpu.sync_copy(idx_hbm.at[pl.ds(base, ROWS_PER_W)], idx_v)

    def gather_descs(c, b):
        return [
            pltpu.make_async_copy(
                table_hbm.at[idx_v.at[pl.ds(c * CHUNK + off, n)]],
                gbuf.at[b, pl.ds(off, n)],
                gsem[b],
            )
            for off, n in GATHER_SPLITS
        ]

    def store_desc(c, b):
        return pltpu.make_async_copy(
            gbuf.at[b], out_hbm.at[pl.ds(base + c * CHUNK, CHUNK)], ssem[b])

    for b in range(NBUF):
        for d in gather_descs(b, b):
            d.start()

    for c in range(N_CHUNKS):
        b = c % NBUF
        for d in gather_descs(c, b):
            d.wait()

        if c >= NBUF:
            store_desc(c - NBUF, b).wait()

        store_desc(c, b).start()

        if c + NBUF < N_CHUNKS:
            for d in gather_descs(c + NBUF, b):
                d.start()

    for b in range(NBUF):
        store_desc(N_CHUNKS - NBUF + b, b).wait()


def _compact_body(raw_ref, out_ref):
    x = raw_ref[...]
    out_ref[...] = lax.slice(x, (0, 0, 0), (BB, SEQ, D)) * SCALE


BB = 32
_STEPS = PART_B // BB


def _compact(k, f, raw):
    if f is None:
        in_specs = [pl.BlockSpec((BB, SEQ_PAD, D), lambda i: (i, 0, 0))]
        args = (raw,)
        aliases = {}
    else:
        in_specs = [
            pl.BlockSpec(memory_space=pl.ANY),
            pl.BlockSpec((BB, SEQ_PAD, D), lambda i: (i, 0, 0)),
        ]
        args = (f, raw)
        aliases = {0: 0}

    def body(*refs):
        _compact_body(refs[-2], refs[-1])

    return pl.pallas_call(
        body,
        grid=(_STEPS,),
        in_specs=in_specs,
        out_specs=pl.BlockSpec(
            (BB, SEQ, D), lambda i, _k=k: (_k * _STEPS + i, 0, 0)),
        out_shape=jax.ShapeDtypeStruct((BATCH, SEQ, D), jnp.float32),
        input_output_aliases=aliases,
    )(*args)


def kernel(input_ids, embed_weight):
    ids = input_ids.astype(jnp.int32)
    idx_pad = jnp.pad(ids, ((0, 0), (0, SEQ_PAD - SEQ))).reshape(
        BATCH * SEQ_PAD)
    raws = []
    for k in range(K_PARTS):
        part_idx = lax.slice(
            idx_pad,
            (k * PART_B * SEQ_PAD,),
            ((k + 1) * PART_B * SEQ_PAD,),
        )
        raw2d = _gather_part(part_idx, embed_weight)
        raws.append(raw2d.reshape(PART_B, SEQ_PAD, D))
    f = _compact(0, None, raws[0])
    for k in range(1, K_PARTS):
        f = _compact(k, f, raws[k])
    return f


# speedup vs baseline: 1.0381x; 1.0381x over previous
import functools
import math

import jax
import jax.numpy as jnp
from jax import lax
from jax.experimental import pallas as pl
from jax.experimental.pallas import tpu as pltpu
from jax.experimental.pallas import tpu_sc as plsc

VOCAB = 100000
D = 128
BATCH = 4096
SEQ = 50
SEQ_PAD = 56
NW = 32
K_PARTS = 4
PART_B = BATCH // K_PARTS
B_PER_W = PART_B // NW
ROWS_PER_W = B_PER_W * SEQ_PAD
NB = 4
CHUNK = NB * SEQ_PAD
GATHER_SPLITS = ((0, 128), (128, 96))
N_CHUNKS = B_PER_W // NB
NBUF = 4
SCALE = math.sqrt(D)

_mesh = plsc.VectorSubcoreMesh(core_axis_name="c", subcore_axis_name="s")


@functools.partial(
    pl.kernel,
    out_type=jax.ShapeDtypeStruct((PART_B * SEQ_PAD, D), jnp.float32),
    mesh=_mesh,
    scratch_types=[
        pltpu.VMEM((ROWS_PER_W,), jnp.int32),
        pltpu.VMEM((NBUF, CHUNK, D), jnp.float32),
    ]
    + [pltpu.SemaphoreType.DMA] * (2 * NBUF),
)
def _gather_part(idx_hbm, table_hbm, out_hbm, idx_v, gbuf, *sems):
    gsem = sems[:NBUF]
    ssem = sems[NBUF:]
    wid = lax.axis_index("s") * 2 + lax.axis_index("c")
    base = wid * ROWS_PER_W
    pltpu.sync_copy(idx_hbm.at[pl.ds(base, ROWS_PER_W)], idx_v)

    def gather_descs(c, b):
        return [
            pltpu.make_async_copy(
                table_hbm.at[idx_v.at[pl.ds(c * CHUNK + off, n)]],
                gbuf.at[b, pl.ds(off, n)],
                gsem[b],
            )
            for off, n in GATHER_SPLITS
        ]

    def store_desc(c, b):
        return pltpu.make_async_copy(
            gbuf.at[b], out_hbm.at[pl.ds(base + c * CHUNK, CHUNK)], ssem[b])

    for b in range(NBUF):
        for d in gather_descs(b, b):
            d.start()

    for c in range(N_CHUNKS):
        b = c % NBUF
        for d in gather_descs(c, b):
            d.wait()

        if c >= NBUF:
            store_desc(c - NBUF, b).wait()

        store_desc(c, b).start()

        if c + NBUF < N_CHUNKS:
            for d in gather_descs(c + NBUF, b):
                d.start()

    for b in range(NBUF):
        store_desc(N_CHUNKS - NBUF + b, b).wait()


def kernel(input_ids, embed_weight):
    ids = input_ids.astype(jnp.int32)
    idx_pad = jnp.pad(ids, ((0, 0), (0, SEQ_PAD - SEQ))).reshape(
        BATCH * SEQ_PAD)
    f = jnp.zeros((BATCH, SEQ, D), jnp.float32)
    for k in range(K_PARTS):
        part_idx = lax.slice(
            idx_pad,
            (k * PART_B * SEQ_PAD,),
            ((k + 1) * PART_B * SEQ_PAD,),
        )
        raw2d = _gather_part(part_idx, embed_weight)
        part = raw2d.reshape(PART_B, SEQ_PAD, D)[:, :SEQ, :] * SCALE
        f = lax.dynamic_update_slice(f, part, (k * PART_B, 0, 0))
    return f
